# baseline (speedup 1.0000x reference)
import functools

import jax
import jax.numpy as jnp
from jax import lax
from jax.experimental import pallas as pl
from jax.experimental.pallas import tpu as pltpu
from jax.experimental.pallas import tpu_sc as plsc

D_MODEL = 64
DH = 32
NUM_CORES = 2
NUM_SUBCORES = 16
NW = NUM_CORES * NUM_SUBCORES
IDX_ROW = 128
CHUNK = 512
STREAMS_PER_CHUNK = CHUNK // IDX_ROW


@functools.lru_cache(maxsize=None)
def _build(flat_n: int):
    b_per_w = flat_n // NW
    n_chunks = b_per_w // CHUNK
    idx_rows_w = b_per_w // IDX_ROW
    assert flat_n % (NW * CHUNK) == 0 and n_chunks % 2 == 0

    mesh = plsc.VectorSubcoreMesh(
        core_axis_name="c", subcore_axis_name="s",
        num_cores=NUM_CORES, num_subcores=NUM_SUBCORES,
    )

    @functools.partial(
        pl.kernel,
        out_type=jax.ShapeDtypeStruct((flat_n, D_MODEL), jnp.float32),
        mesh=mesh,
        compiler_params=pltpu.CompilerParams(use_tc_tiling_on_sc=False),
        scratch_types=[
            pltpu.VMEM((idx_rows_w, IDX_ROW), jnp.int32),
            pltpu.VMEM((CHUNK, DH), jnp.float32),
            pltpu.VMEM((CHUNK, DH), jnp.float32),
            pltpu.VMEM((CHUNK, DH), jnp.float32),
            pltpu.VMEM((CHUNK, DH), jnp.float32),
            pltpu.SemaphoreType.DMA,
            pltpu.SemaphoreType.DMA,
            pltpu.SemaphoreType.DMA,
            pltpu.SemaphoreType.DMA,
        ],
    )
    def gather_k(idx_hbm, ta_hbm, tb_hbm, out_hbm,
                 idx_v, ra0, rb0, ra1, rb1, gsem0, gsem1, osem0, osem1):
        wid = lax.axis_index("s") * NUM_CORES + lax.axis_index("c")
        base = wid * b_per_w
        row0 = wid * idx_rows_w
        pltpu.sync_copy(idx_hbm.at[pl.ds(row0, idx_rows_w)], idx_v)

        rowsa = (ra0, ra1)
        rowsb = (rb0, rb1)
        gsem = (gsem0, gsem1)
        osem = (osem0, osem1)

        def fire_gathers(i, b):
            for j in range(STREAMS_PER_CHUNK):
                idx_ref = idx_v.at[i * STREAMS_PER_CHUNK + j]
                pltpu.async_copy(
                    ta_hbm.at[idx_ref],
                    rowsa[b].at[pl.ds(j * IDX_ROW, IDX_ROW)],
                    gsem[b],
                )
                pltpu.async_copy(
                    tb_hbm.at[idx_ref],
                    rowsb[b].at[pl.ds(j * IDX_ROW, IDX_ROW)],
                    gsem[b],
                )

        def drain_gathers(b):
            pltpu.make_async_copy(
                ta_hbm.at[pl.ds(0, CHUNK)], rowsa[b], gsem[b]
            ).wait()
            pltpu.make_async_copy(
                tb_hbm.at[pl.ds(0, CHUNK)], rowsb[b], gsem[b]
            ).wait()

        def start_write(i, b):
            pltpu.async_copy(
                rowsa[b], out_hbm.at[pl.ds(base + i * CHUNK, CHUNK), pl.ds(0, DH)],
                osem[b],
            )
            pltpu.async_copy(
                rowsb[b], out_hbm.at[pl.ds(base + i * CHUNK, CHUNK), pl.ds(DH, DH)],
                osem[b],
            )

        def drain_write(b):
            pltpu.make_async_copy(
                rowsa[b], out_hbm.at[pl.ds(base, CHUNK), pl.ds(0, DH)], osem[b]
            ).wait()
            pltpu.make_async_copy(
                rowsb[b], out_hbm.at[pl.ds(base, CHUNK), pl.ds(DH, DH)], osem[b]
            ).wait()

        def body(t, carry):
            for b in range(2):
                i = 2 * t + b
                pb = 1 - b
                @pl.when(t >= 1)
                def _():
                    drain_write(b)
                fire_gathers(i, b)
                @pl.when(i >= 1)
                def _():
                    drain_gathers(pb)
                    start_write(i - 1, pb)
            return carry

        lax.fori_loop(0, n_chunks // 2, body, 0)
        last = n_chunks - 1
        drain_gathers(1)
        start_write(last, 1)
        drain_write(0)
        drain_write(1)

    return gather_k


def kernel(token_ids, weight):
    batch, hist = token_ids.shape
    flat = token_ids.reshape(-1).astype(jnp.int32)
    idx2d = flat.reshape(-1, IDX_ROW)
    ta = weight[:, :DH]
    tb = weight[:, DH:]
    out = _build(flat.shape[0])(idx2d, ta, tb)
    return out.reshape(batch, hist, D_MODEL)


# speedup vs baseline: 1.6317x; 1.6317x over previous
import functools

import jax
import jax.numpy as jnp
from jax import lax
from jax.experimental import pallas as pl
from jax.experimental.pallas import tpu as pltpu
from jax.experimental.pallas import tpu_sc as plsc

D_MODEL = 64
NUM_CORES = 2
NUM_SUBCORES = 16
NW = NUM_CORES * NUM_SUBCORES
IDX_ROW = 128
CHUNK = 512
STREAMS_PER_CHUNK = CHUNK // IDX_ROW


@functools.lru_cache(maxsize=None)
def _build(flat_n: int, vocab: int):
    b_per_w = flat_n // NW
    n_chunks = b_per_w // CHUNK
    idx_rows_w = b_per_w // IDX_ROW
    assert flat_n % (NW * CHUNK) == 0 and n_chunks % 2 == 0

    mesh = plsc.VectorSubcoreMesh(
        core_axis_name="c", subcore_axis_name="s",
        num_cores=NUM_CORES, num_subcores=NUM_SUBCORES,
    )

    @functools.partial(
        pl.kernel,
        out_type=jax.ShapeDtypeStruct((flat_n, D_MODEL), jnp.float32),
        mesh=mesh,
        compiler_params=pltpu.CompilerParams(use_tc_tiling_on_sc=False),
        scratch_types=[
            pltpu.VMEM((idx_rows_w, IDX_ROW), jnp.int32),
            pltpu.VMEM((CHUNK, D_MODEL), jnp.float32),
            pltpu.VMEM((CHUNK, D_MODEL), jnp.float32),
            pltpu.SemaphoreType.DMA,
            pltpu.SemaphoreType.DMA,
            pltpu.SemaphoreType.DMA,
            pltpu.SemaphoreType.DMA,
        ],
    )
    def gather_k(idx_hbm, table_hbm, out_hbm,
                 idx_v, rows0, rows1, gsem0, gsem1, osem0, osem1):
        wid = lax.axis_index("s") * NUM_CORES + lax.axis_index("c")
        base = wid * b_per_w
        row0 = wid * idx_rows_w
        pltpu.sync_copy(idx_hbm.at[pl.ds(row0, idx_rows_w)], idx_v)

        rows = (rows0, rows1)
        gsem = (gsem0, gsem1)
        osem = (osem0, osem1)

        def fire_gathers(i, b):
            for j in range(STREAMS_PER_CHUNK):
                pltpu.async_copy(
                    table_hbm.at[idx_v.at[i * STREAMS_PER_CHUNK + j]],
                    rows[b].at[pl.ds(j * IDX_ROW, IDX_ROW)],
                    gsem[b],
                )

        def drain_gathers(b):
            pltpu.make_async_copy(
                table_hbm.at[pl.ds(0, CHUNK)], rows[b], gsem[b]
            ).wait()

        def drain_write(b):
            pltpu.make_async_copy(
                rows[b], out_hbm.at[pl.ds(base, CHUNK)], osem[b]
            ).wait()

        def body(t, carry):
            for b in range(2):
                i = 2 * t + b
                pb = 1 - b
                @pl.when(t >= 1)
                def _():
                    drain_write(b)
                fire_gathers(i, b)
                @pl.when(i >= 1)
                def _():
                    drain_gathers(pb)
                    pltpu.async_copy(
                        rows[pb],
                        out_hbm.at[pl.ds(base + (i - 1) * CHUNK, CHUNK)],
                        osem[pb],
                    )
            return carry

        lax.fori_loop(0, n_chunks // 2, body, 0)
        last = n_chunks - 1
        drain_gathers(1)
        pltpu.async_copy(
            rows[1], out_hbm.at[pl.ds(base + last * CHUNK, CHUNK)], osem[1]
        )
        drain_write(0)
        drain_write(1)

    return gather_k


def kernel(token_ids, weight):
    batch, hist = token_ids.shape
    vocab, d = weight.shape
    flat = token_ids.reshape(-1).astype(jnp.int32)
    idx2d = flat.reshape(-1, IDX_ROW)
    out = _build(flat.shape[0], vocab)(idx2d, weight)
    return out.reshape(batch, hist, d)
